# Initial kernel scaffold; baseline (speedup 1.0000x reference)
import jax
import jax.numpy as jnp
from jax.experimental import pallas as pl


def kernel(anchors, valid_flags, gt_bboxes):
    raise NotImplementedError("write your pallas kernel here")


# speedup vs baseline: 2.8256x; 2.8256x over previous
import jax
import jax.numpy as jnp
import numpy as np
from jax.experimental import pallas as pl
from jax.experimental.pallas import tpu as pltpu

_N = 20000
_G = 100
_IMG = 1344.0
_ROWS = 160
_LANES = 128
_NP = _ROWS * _LANES
_K = 128


def _make_ranks():
    kp, kn = jax.random.split(jax.random.key(42))
    out = []
    for k in (kp, kn):
        pri = np.asarray(jax.random.uniform(k, (_N,)))
        perm = np.argsort(-pri, kind="stable")
        rank = np.empty(_N, np.int32)
        rank[perm] = np.arange(_N, dtype=np.int32)
        pad = np.full(_NP - _N, np.int32(1 << 30), np.int32)
        out.append(np.concatenate([rank, pad]).reshape(_ROWS, _LANES))
    return out[0], out[1]


_RANK_POS, _RANK_NEG = _make_ranks()


def _body(gt_ref, a_ref, v_ref, rp_ref, rn_ref,
          lab_ref, lw_ref, posf_ref, tgt_ref, npos_ref, nneg_ref,
          maxov_s, argm_s, lqgt_s, mx1_s, my1_s, mx2_s, my2_s):
    g = pl.program_id(0)

    ax1 = a_ref[0]
    ay1 = a_ref[1]
    ax2 = a_ref[2]
    ay2 = a_ref[3]

    @pl.when(g < _G)
    def _():
        gx1 = gt_ref[0, g]
        gy1 = gt_ref[1, g]
        gx2 = gt_ref[2, g]
        gy2 = gt_ref[3, g]
        a1 = (ax2 - ax1 + 1.0) * (ay2 - ay1 + 1.0)
        a2 = (gx2 - gx1 + 1.0) * (gy2 - gy1 + 1.0)
        wx = jnp.maximum(jnp.minimum(ax2, gx2) - jnp.maximum(ax1, gx1) + 1.0, 0.0)
        wy = jnp.maximum(jnp.minimum(ay2, gy2) - jnp.maximum(ay1, gy1) + 1.0, 0.0)
        inter = wx * wy
        iou = inter / (a1 + a2 - inter)
        gmax = jnp.max(iou)
        lqf = (iou >= gmax - 1e-6) & (gmax >= 0.3)

        @pl.when(g == 0)
        def _():
            maxov_s[...] = iou
            argm_s[...] = jnp.zeros_like(argm_s)
            lqgt_s[...] = jnp.where(lqf, 0, -1)

        @pl.when(g > 0)
        def _():
            prev = maxov_s[...]
            better = iou > prev
            maxov_s[...] = jnp.where(better, iou, prev)
            argm_s[...] = jnp.where(better, g, argm_s[...])
            lqgt_s[...] = jnp.where(lqf, g, lqgt_s[...])

    @pl.when(g == _G)
    def _():
        inside = ((v_ref[...] != 0) & (ax1 >= 0.0) & (ay1 >= 0.0)
                  & (ax2 < _IMG) & (ay2 < _IMG))
        maxov = maxov_s[...]
        argm = argm_s[...]
        lqgt = lqgt_s[...]
        assigned = jnp.where((maxov >= -1.0) & (maxov < 0.3), 0, -1)
        assigned = jnp.where(maxov >= 0.7, argm + 1, assigned)
        assigned = jnp.where(lqgt >= 0, lqgt + 1, assigned)
        assigned = jnp.where(inside, assigned, -1)
        pos_m = assigned > 0
        neg_m = assigned == 0

        def search(mask, rank):
            def bsb(_, lh):
                lo, hi = lh
                mid = (lo + hi) // 2
                cnt = jnp.sum(jnp.where(mask & (rank <= mid), 1, 0))
                ge = cnt >= _K
                nlo = jnp.where(ge, lo, mid + 1)
                nhi = jnp.where(ge, mid, hi)
                cont = lo < hi
                return (jnp.where(cont, nlo, lo), jnp.where(cont, nhi, hi))
            lo, _hi = jax.lax.fori_loop(
                0, 16, bsb, (jnp.int32(0), jnp.int32(32768)))
            return lo

        tp = search(pos_m, rp_ref[...])
        tn = search(neg_m, rn_ref[...])
        sp = pos_m & (rp_ref[...] <= tp)
        sn = neg_m & (rn_ref[...] <= tn)

        lab_ref[...] = jnp.where(sp, 1, 0)
        lw_ref[...] = jnp.where(sp | sn, 1.0, 0.0)
        posf_ref[...] = jnp.where(sp, 1.0, 0.0)
        npos_ref[0, 0] = jnp.sum(jnp.where(sp, 1, 0))
        nneg_ref[0, 0] = jnp.sum(jnp.where(sn, 1, 0))

        gidx = jnp.where(lqgt >= 0, lqgt, argm)

        def mgb(j, c):
            m = gidx == j
            mx1_s[...] = jnp.where(m, gt_ref[0, j], mx1_s[...])
            my1_s[...] = jnp.where(m, gt_ref[1, j], my1_s[...])
            mx2_s[...] = jnp.where(m, gt_ref[2, j], mx2_s[...])
            my2_s[...] = jnp.where(m, gt_ref[3, j], my2_s[...])
            return c
        jax.lax.fori_loop(0, _G, mgb, 0)

        mx1 = mx1_s[...]
        my1 = my1_s[...]
        mx2 = mx2_s[...]
        my2 = my2_s[...]
        px = (ax1 + ax2) * 0.5
        py = (ay1 + ay2) * 0.5
        pw = ax2 - ax1 + 1.0
        ph = ay2 - ay1 + 1.0
        gx = (mx1 + mx2) * 0.5
        gy = (my1 + my2) * 0.5
        gw = mx2 - mx1 + 1.0
        gh = my2 - my1 + 1.0
        tgt_ref[0] = jnp.where(sp, (gx - px) / pw, 0.0)
        tgt_ref[1] = jnp.where(sp, (gy - py) / ph, 0.0)
        tgt_ref[2] = jnp.where(sp, jnp.log(gw / pw), 0.0)
        tgt_ref[3] = jnp.where(sp, jnp.log(gh / ph), 0.0)


def _run(a4, v2, gt4, rp, rn):
    f32 = jnp.float32
    i32 = jnp.int32
    vmem2 = pl.BlockSpec((_ROWS, _LANES), lambda g: (0, 0))
    return pl.pallas_call(
        _body,
        grid=(_G + 1,),
        in_specs=[
            pl.BlockSpec((4, _G), lambda g: (0, 0), memory_space=pltpu.SMEM),
            pl.BlockSpec((4, _ROWS, _LANES), lambda g: (0, 0, 0)),
            vmem2,
            vmem2,
            vmem2,
        ],
        out_specs=[
            vmem2,
            vmem2,
            vmem2,
            pl.BlockSpec((4, _ROWS, _LANES), lambda g: (0, 0, 0)),
            pl.BlockSpec((1, 1), lambda g: (0, 0), memory_space=pltpu.SMEM),
            pl.BlockSpec((1, 1), lambda g: (0, 0), memory_space=pltpu.SMEM),
        ],
        out_shape=[
            jax.ShapeDtypeStruct((_ROWS, _LANES), i32),
            jax.ShapeDtypeStruct((_ROWS, _LANES), f32),
            jax.ShapeDtypeStruct((_ROWS, _LANES), f32),
            jax.ShapeDtypeStruct((4, _ROWS, _LANES), f32),
            jax.ShapeDtypeStruct((1, 1), i32),
            jax.ShapeDtypeStruct((1, 1), i32),
        ],
        scratch_shapes=[
            pltpu.VMEM((_ROWS, _LANES), f32),
            pltpu.VMEM((_ROWS, _LANES), i32),
            pltpu.VMEM((_ROWS, _LANES), i32),
            pltpu.VMEM((_ROWS, _LANES), f32),
            pltpu.VMEM((_ROWS, _LANES), f32),
            pltpu.VMEM((_ROWS, _LANES), f32),
            pltpu.VMEM((_ROWS, _LANES), f32),
        ],
    )(gt4, a4, v2, rp, rn)


def kernel(anchors, valid_flags, gt_bboxes):
    pad_box = jnp.array([-1e6, -1e6, -1e6 + 100.0, -1e6 + 100.0], jnp.float32)
    a_p = jnp.concatenate(
        [anchors, jnp.broadcast_to(pad_box, (_NP - _N, 4))], axis=0)
    a4 = a_p.T.reshape(4, _ROWS, _LANES)
    v2 = jnp.concatenate(
        [valid_flags.astype(jnp.int32),
         jnp.zeros((_NP - _N,), jnp.int32)]).reshape(_ROWS, _LANES)
    gt4 = gt_bboxes.T
    rp = jnp.asarray(_RANK_POS)
    rn = jnp.asarray(_RANK_NEG)

    lab, lw, posf, tgt, npos, nneg = _run(a4, v2, gt4, rp, rn)

    labels = lab.reshape(-1)[:_N]
    label_weights = lw.reshape(-1)[:_N]
    bbox_targets = tgt.reshape(4, -1)[:, :_N].T
    posf1 = posf.reshape(-1)[:_N]
    bbox_weights = jnp.broadcast_to(posf1[:, None], (_N, 4))
    num_pos = npos[0, 0]
    num_neg = nneg[0, 0]
    return labels, label_weights, bbox_targets, bbox_weights, num_pos, num_neg


# speedup vs baseline: 5.1297x; 1.8154x over previous
import jax
import jax.numpy as jnp
import numpy as np
from jax.experimental import pallas as pl
from jax.experimental.pallas import tpu as pltpu

_N = 20000
_G = 100
_IMG = 1344.0
_ROWS = 160
_LANES = 128
_NP = _ROWS * _LANES
_K = 128


def _make_ranks():
    kp, kn = jax.random.split(jax.random.key(42))
    out = []
    for k in (kp, kn):
        pri = np.asarray(jax.random.uniform(k, (_N,)))
        perm = np.argsort(-pri, kind="stable")
        rank = np.empty(_N, np.int32)
        rank[perm] = np.arange(_N, dtype=np.int32)
        pad = np.full(_NP - _N, np.int32(1 << 30), np.int32)
        out.append(np.concatenate([rank, pad]).reshape(_ROWS, _LANES))
    return out[0], out[1]


_RANK_POS, _RANK_NEG = _make_ranks()


_UNROLL = 4
_NSTEPS = _G // _UNROLL


def _body(gt_ref, a_ref, v_ref, rp_ref, rn_ref,
          lab_ref, lw_ref, posf_ref, tgt_ref, npos_ref, nneg_ref,
          maxov_s, argm_s, lqgt_s):
    s = pl.program_id(0)

    ax1 = a_ref[0]
    ay1 = a_ref[1]
    ax2 = a_ref[2]
    ay2 = a_ref[3]

    @pl.when(s < _NSTEPS)
    def _():
        a1 = (ax2 - ax1 + 1.0) * (ay2 - ay1 + 1.0)
        first = s == 0
        mo = jnp.where(first, jnp.float32(-jnp.inf), maxov_s[...])
        am = jnp.where(first, 0, argm_s[...])
        lq = jnp.where(first, -1, lqgt_s[...])
        for j in range(_UNROLL):
            g = s * _UNROLL + j
            gx1 = gt_ref[0, g]
            gy1 = gt_ref[1, g]
            gx2 = gt_ref[2, g]
            gy2 = gt_ref[3, g]
            a2 = (gx2 - gx1 + 1.0) * (gy2 - gy1 + 1.0)
            wx = jnp.maximum(
                jnp.minimum(ax2, gx2) - jnp.maximum(ax1, gx1) + 1.0, 0.0)
            wy = jnp.maximum(
                jnp.minimum(ay2, gy2) - jnp.maximum(ay1, gy1) + 1.0, 0.0)
            inter = wx * wy
            iou = inter / (a1 + a2 - inter)
            gmax = jnp.max(iou)
            lqf = (iou >= gmax - 1e-6) & (gmax >= 0.3)
            better = iou > mo
            mo = jnp.where(better, iou, mo)
            am = jnp.where(better, g, am)
            lq = jnp.where(lqf, g, lq)
        maxov_s[...] = mo
        argm_s[...] = am
        lqgt_s[...] = lq

    @pl.when(s == _NSTEPS)
    def _():
        inside = ((v_ref[...] != 0) & (ax1 >= 0.0) & (ay1 >= 0.0)
                  & (ax2 < _IMG) & (ay2 < _IMG))
        maxov = maxov_s[...]
        argm = argm_s[...]
        lqgt = lqgt_s[...]
        assigned = jnp.where((maxov >= -1.0) & (maxov < 0.3), 0, -1)
        assigned = jnp.where(maxov >= 0.7, argm + 1, assigned)
        assigned = jnp.where(lqgt >= 0, lqgt + 1, assigned)
        assigned = jnp.where(inside, assigned, -1)
        pos_m = assigned > 0
        neg_m = assigned == 0

        rp = rp_ref[...]
        rn = rn_ref[...]

        def bsb(_, st):
            plo, phi, nlo, nhi = st
            pmid = (plo + phi) // 2
            nmid = (nlo + nhi) // 2
            pcnt = jnp.sum(jnp.where(pos_m & (rp <= pmid), 1, 0))
            ncnt = jnp.sum(jnp.where(neg_m & (rn <= nmid), 1, 0))
            pge = pcnt >= _K
            nge = ncnt >= _K
            pc = plo < phi
            nc = nlo < nhi
            return (jnp.where(pc & pge, plo, jnp.where(pc, pmid + 1, plo)),
                    jnp.where(pc & pge, pmid, phi),
                    jnp.where(nc & nge, nlo, jnp.where(nc, nmid + 1, nlo)),
                    jnp.where(nc & nge, nmid, nhi))

        z = jnp.int32(0)
        top = jnp.int32(32768)
        tp, _, tn, _ = jax.lax.fori_loop(0, 16, bsb, (z, top, z, top))
        sp = pos_m & (rp <= tp)
        sn = neg_m & (rn <= tn)

        lab_ref[...] = jnp.where(sp, 1, 0)
        lw_ref[...] = jnp.where(sp | sn, 1.0, 0.0)
        posf_ref[...] = jnp.where(sp, 1.0, 0.0)
        npos_ref[0, 0] = jnp.sum(jnp.where(sp, 1, 0))
        nneg_ref[0, 0] = jnp.sum(jnp.where(sn, 1, 0))

        gidx = jnp.where(lqgt >= 0, lqgt, argm)

        def mgb(i, c):
            x1, y1, x2, y2 = c
            for k in range(_UNROLL):
                j = i * _UNROLL + k
                m = gidx == j
                x1 = jnp.where(m, gt_ref[0, j], x1)
                y1 = jnp.where(m, gt_ref[1, j], y1)
                x2 = jnp.where(m, gt_ref[2, j], x2)
                y2 = jnp.where(m, gt_ref[3, j], y2)
            return (x1, y1, x2, y2)

        zf = jnp.zeros_like(maxov)
        mx1, my1, mx2, my2 = jax.lax.fori_loop(
            0, _G // _UNROLL, mgb, (zf, zf, zf, zf))
        px = (ax1 + ax2) * 0.5
        py = (ay1 + ay2) * 0.5
        pw = ax2 - ax1 + 1.0
        ph = ay2 - ay1 + 1.0
        gx = (mx1 + mx2) * 0.5
        gy = (my1 + my2) * 0.5
        gw = mx2 - mx1 + 1.0
        gh = my2 - my1 + 1.0
        tgt_ref[0] = jnp.where(sp, (gx - px) / pw, 0.0)
        tgt_ref[1] = jnp.where(sp, (gy - py) / ph, 0.0)
        tgt_ref[2] = jnp.where(sp, jnp.log(gw / pw), 0.0)
        tgt_ref[3] = jnp.where(sp, jnp.log(gh / ph), 0.0)


def _run(a4, v2, gt4, rp, rn):
    f32 = jnp.float32
    i32 = jnp.int32
    vmem2 = pl.BlockSpec((_ROWS, _LANES), lambda g: (0, 0))
    return pl.pallas_call(
        _body,
        grid=(_NSTEPS + 1,),
        in_specs=[
            pl.BlockSpec((4, _G), lambda g: (0, 0), memory_space=pltpu.SMEM),
            pl.BlockSpec((4, _ROWS, _LANES), lambda g: (0, 0, 0)),
            vmem2,
            vmem2,
            vmem2,
        ],
        out_specs=[
            vmem2,
            vmem2,
            vmem2,
            pl.BlockSpec((4, _ROWS, _LANES), lambda g: (0, 0, 0)),
            pl.BlockSpec((1, 1), lambda g: (0, 0), memory_space=pltpu.SMEM),
            pl.BlockSpec((1, 1), lambda g: (0, 0), memory_space=pltpu.SMEM),
        ],
        out_shape=[
            jax.ShapeDtypeStruct((_ROWS, _LANES), i32),
            jax.ShapeDtypeStruct((_ROWS, _LANES), f32),
            jax.ShapeDtypeStruct((_ROWS, _LANES), f32),
            jax.ShapeDtypeStruct((4, _ROWS, _LANES), f32),
            jax.ShapeDtypeStruct((1, 1), i32),
            jax.ShapeDtypeStruct((1, 1), i32),
        ],
        scratch_shapes=[
            pltpu.VMEM((_ROWS, _LANES), f32),
            pltpu.VMEM((_ROWS, _LANES), i32),
            pltpu.VMEM((_ROWS, _LANES), i32),
        ],
    )(gt4, a4, v2, rp, rn)


def kernel(anchors, valid_flags, gt_bboxes):
    pad_box = jnp.array([-1e6, -1e6, -1e6 + 100.0, -1e6 + 100.0], jnp.float32)
    a_p = jnp.concatenate(
        [anchors, jnp.broadcast_to(pad_box, (_NP - _N, 4))], axis=0)
    a4 = a_p.T.reshape(4, _ROWS, _LANES)
    v2 = jnp.concatenate(
        [valid_flags.astype(jnp.int32),
         jnp.zeros((_NP - _N,), jnp.int32)]).reshape(_ROWS, _LANES)
    gt4 = gt_bboxes.T
    rp = jnp.asarray(_RANK_POS)
    rn = jnp.asarray(_RANK_NEG)

    lab, lw, posf, tgt, npos, nneg = _run(a4, v2, gt4, rp, rn)

    labels = lab.reshape(-1)[:_N]
    label_weights = lw.reshape(-1)[:_N]
    bbox_targets = tgt.reshape(4, -1)[:, :_N].T
    posf1 = posf.reshape(-1)[:_N]
    bbox_weights = jnp.broadcast_to(posf1[:, None], (_N, 4))
    num_pos = npos[0, 0]
    num_neg = nneg[0, 0]
    return labels, label_weights, bbox_targets, bbox_weights, num_pos, num_neg


# speedup vs baseline: 5.7487x; 1.1207x over previous
import jax
import jax.numpy as jnp
import numpy as np
from jax.experimental import pallas as pl
from jax.experimental.pallas import tpu as pltpu

_N = 20000
_G = 100
_IMG = 1344.0
_ROWS = 160
_LANES = 128
_NP = _ROWS * _LANES
_K = 128


def _make_ranks():
    kp, kn = jax.random.split(jax.random.key(42))
    out = []
    for k in (kp, kn):
        pri = np.asarray(jax.random.uniform(k, (_N,)))
        perm = np.argsort(-pri, kind="stable")
        rank = np.empty(_N, np.int32)
        rank[perm] = np.arange(_N, dtype=np.int32)
        pad = np.full(_NP - _N, np.int32(1 << 30), np.int32)
        out.append(np.concatenate([rank, pad]).reshape(_ROWS, _LANES))
    return out[0], out[1]


_RANK_POS, _RANK_NEG = _make_ranks()


_UNROLL = 10
_NSTEPS = _G // _UNROLL


def _body(gt_ref, a_ref, v_ref, rp_ref, rn_ref,
          lab_ref, lw_ref, posf_ref, tgt_ref, npos_ref, nneg_ref,
          maxov_s, argm_s, lqgt_s):
    s = pl.program_id(0)

    ax1 = a_ref[0]
    ay1 = a_ref[1]
    ax2 = a_ref[2]
    ay2 = a_ref[3]

    @pl.when(s < _NSTEPS)
    def _():
        a1 = (ax2 - ax1 + 1.0) * (ay2 - ay1 + 1.0)
        first = s == 0
        mo = jnp.where(first, jnp.float32(-jnp.inf), maxov_s[...])
        am = jnp.where(first, 0, argm_s[...])
        lq = jnp.where(first, -1, lqgt_s[...])
        for j in range(_UNROLL):
            g = s * _UNROLL + j
            gx1 = gt_ref[0, g]
            gy1 = gt_ref[1, g]
            gx2 = gt_ref[2, g]
            gy2 = gt_ref[3, g]
            a2 = (gx2 - gx1 + 1.0) * (gy2 - gy1 + 1.0)
            wx = jnp.maximum(
                jnp.minimum(ax2, gx2) - jnp.maximum(ax1, gx1) + 1.0, 0.0)
            wy = jnp.maximum(
                jnp.minimum(ay2, gy2) - jnp.maximum(ay1, gy1) + 1.0, 0.0)
            inter = wx * wy
            iou = inter / (a1 + a2 - inter)
            gmax = jnp.max(iou)
            lqf = (iou >= gmax - 1e-6) & (gmax >= 0.3)
            better = iou > mo
            mo = jnp.where(better, iou, mo)
            am = jnp.where(better, g, am)
            lq = jnp.where(lqf, g, lq)
        maxov_s[...] = mo
        argm_s[...] = am
        lqgt_s[...] = lq

    @pl.when(s == _NSTEPS)
    def _():
        inside = ((v_ref[...] != 0) & (ax1 >= 0.0) & (ay1 >= 0.0)
                  & (ax2 < _IMG) & (ay2 < _IMG))
        maxov = maxov_s[...]
        argm = argm_s[...]
        lqgt = lqgt_s[...]
        assigned = jnp.where((maxov >= -1.0) & (maxov < 0.3), 0, -1)
        assigned = jnp.where(maxov >= 0.7, argm + 1, assigned)
        assigned = jnp.where(lqgt >= 0, lqgt + 1, assigned)
        assigned = jnp.where(inside, assigned, -1)
        pos_m = assigned > 0
        neg_m = assigned == 0

        rp = rp_ref[...]
        rn = rn_ref[...]

        def bsb(_, st):
            plo, phi, nlo, nhi = st
            pmid = (plo + phi) // 2
            nmid = (nlo + nhi) // 2
            pcnt = jnp.sum(jnp.where(pos_m & (rp <= pmid), 1, 0))
            ncnt = jnp.sum(jnp.where(neg_m & (rn <= nmid), 1, 0))
            pge = pcnt >= _K
            nge = ncnt >= _K
            pc = plo < phi
            nc = nlo < nhi
            return (jnp.where(pc & pge, plo, jnp.where(pc, pmid + 1, plo)),
                    jnp.where(pc & pge, pmid, phi),
                    jnp.where(nc & nge, nlo, jnp.where(nc, nmid + 1, nlo)),
                    jnp.where(nc & nge, nmid, nhi))

        z = jnp.int32(0)
        top = jnp.int32(32768)
        tp, _, tn, _ = jax.lax.fori_loop(0, 16, bsb, (z, top, z, top))
        sp = pos_m & (rp <= tp)
        sn = neg_m & (rn <= tn)

        lab_ref[...] = jnp.where(sp, 1, 0)
        lw_ref[...] = jnp.where(sp | sn, 1.0, 0.0)
        posf_ref[...] = jnp.where(sp, 1.0, 0.0)
        npos_ref[0, 0] = jnp.sum(jnp.where(sp, 1, 0))
        nneg_ref[0, 0] = jnp.sum(jnp.where(sn, 1, 0))

        gidx = jnp.where(lqgt >= 0, lqgt, argm)

        def mgb(i, c):
            x1, y1, x2, y2 = c
            for k in range(_UNROLL):
                j = i * _UNROLL + k
                m = gidx == j
                x1 = jnp.where(m, gt_ref[0, j], x1)
                y1 = jnp.where(m, gt_ref[1, j], y1)
                x2 = jnp.where(m, gt_ref[2, j], x2)
                y2 = jnp.where(m, gt_ref[3, j], y2)
            return (x1, y1, x2, y2)

        zf = jnp.zeros_like(maxov)
        mx1, my1, mx2, my2 = jax.lax.fori_loop(
            0, _G // _UNROLL, mgb, (zf, zf, zf, zf))
        px = (ax1 + ax2) * 0.5
        py = (ay1 + ay2) * 0.5
        pw = ax2 - ax1 + 1.0
        ph = ay2 - ay1 + 1.0
        gx = (mx1 + mx2) * 0.5
        gy = (my1 + my2) * 0.5
        gw = mx2 - mx1 + 1.0
        gh = my2 - my1 + 1.0
        tgt_ref[0] = jnp.where(sp, (gx - px) / pw, 0.0)
        tgt_ref[1] = jnp.where(sp, (gy - py) / ph, 0.0)
        tgt_ref[2] = jnp.where(sp, jnp.log(gw / pw), 0.0)
        tgt_ref[3] = jnp.where(sp, jnp.log(gh / ph), 0.0)


def _run(a4, v2, gt4, rp, rn):
    f32 = jnp.float32
    i32 = jnp.int32
    vmem2 = pl.BlockSpec((_ROWS, _LANES), lambda g: (0, 0))
    return pl.pallas_call(
        _body,
        grid=(_NSTEPS + 1,),
        in_specs=[
            pl.BlockSpec((4, _G), lambda g: (0, 0), memory_space=pltpu.SMEM),
            pl.BlockSpec((4, _ROWS, _LANES), lambda g: (0, 0, 0)),
            vmem2,
            vmem2,
            vmem2,
        ],
        out_specs=[
            vmem2,
            vmem2,
            vmem2,
            pl.BlockSpec((4, _ROWS, _LANES), lambda g: (0, 0, 0)),
            pl.BlockSpec((1, 1), lambda g: (0, 0), memory_space=pltpu.SMEM),
            pl.BlockSpec((1, 1), lambda g: (0, 0), memory_space=pltpu.SMEM),
        ],
        out_shape=[
            jax.ShapeDtypeStruct((_ROWS, _LANES), i32),
            jax.ShapeDtypeStruct((_ROWS, _LANES), f32),
            jax.ShapeDtypeStruct((_ROWS, _LANES), f32),
            jax.ShapeDtypeStruct((4, _ROWS, _LANES), f32),
            jax.ShapeDtypeStruct((1, 1), i32),
            jax.ShapeDtypeStruct((1, 1), i32),
        ],
        scratch_shapes=[
            pltpu.VMEM((_ROWS, _LANES), f32),
            pltpu.VMEM((_ROWS, _LANES), i32),
            pltpu.VMEM((_ROWS, _LANES), i32),
        ],
    )(gt4, a4, v2, rp, rn)


def kernel(anchors, valid_flags, gt_bboxes):
    pad_box = jnp.array([-1e6, -1e6, -1e6 + 100.0, -1e6 + 100.0], jnp.float32)
    a_p = jnp.concatenate(
        [anchors, jnp.broadcast_to(pad_box, (_NP - _N, 4))], axis=0)
    a4 = a_p.T.reshape(4, _ROWS, _LANES)
    v2 = jnp.concatenate(
        [valid_flags.astype(jnp.int32),
         jnp.zeros((_NP - _N,), jnp.int32)]).reshape(_ROWS, _LANES)
    gt4 = gt_bboxes.T
    rp = jnp.asarray(_RANK_POS)
    rn = jnp.asarray(_RANK_NEG)

    lab, lw, posf, tgt, npos, nneg = _run(a4, v2, gt4, rp, rn)

    labels = lab.reshape(-1)[:_N]
    label_weights = lw.reshape(-1)[:_N]
    bbox_targets = tgt.reshape(4, -1)[:, :_N].T
    posf1 = posf.reshape(-1)[:_N]
    bbox_weights = jnp.broadcast_to(posf1[:, None], (_N, 4))
    num_pos = npos[0, 0]
    num_neg = nneg[0, 0]
    return labels, label_weights, bbox_targets, bbox_weights, num_pos, num_neg


# speedup vs baseline: 5.8570x; 1.0188x over previous
import jax
import jax.numpy as jnp
import numpy as np
from jax.experimental import pallas as pl
from jax.experimental.pallas import tpu as pltpu

_N = 20000
_G = 100
_IMG = 1344.0
_ROWS = 160
_LANES = 128
_NP = _ROWS * _LANES
_K = 128


def _make_ranks():
    kp, kn = jax.random.split(jax.random.key(42))
    out = []
    for k in (kp, kn):
        pri = np.asarray(jax.random.uniform(k, (_N,)))
        perm = np.argsort(-pri, kind="stable")
        rank = np.empty(_N, np.int32)
        rank[perm] = np.arange(_N, dtype=np.int32)
        pad = np.full(_NP - _N, np.int32(1 << 30), np.int32)
        out.append(np.concatenate([rank, pad]).reshape(_ROWS, _LANES))
    return out[0], out[1]


_RANK_POS, _RANK_NEG = _make_ranks()


_UNROLL = 20
_NSTEPS = _G // _UNROLL


def _body(gt_ref, a_ref, v_ref, rp_ref, rn_ref,
          lab_ref, lw_ref, posf_ref, tgt_ref, npos_ref, nneg_ref,
          maxov_s, argm_s, lqgt_s):
    s = pl.program_id(0)

    ax1 = a_ref[0]
    ay1 = a_ref[1]
    ax2 = a_ref[2]
    ay2 = a_ref[3]

    @pl.when(s < _NSTEPS)
    def _():
        a1 = (ax2 - ax1 + 1.0) * (ay2 - ay1 + 1.0)
        first = s == 0
        mo = jnp.where(first, jnp.float32(-jnp.inf), maxov_s[...])
        am = jnp.where(first, 0, argm_s[...])
        lq = jnp.where(first, -1, lqgt_s[...])
        for j in range(_UNROLL):
            g = s * _UNROLL + j
            gx1 = gt_ref[0, g]
            gy1 = gt_ref[1, g]
            gx2 = gt_ref[2, g]
            gy2 = gt_ref[3, g]
            a2 = (gx2 - gx1 + 1.0) * (gy2 - gy1 + 1.0)
            wx = jnp.maximum(
                jnp.minimum(ax2, gx2) - jnp.maximum(ax1, gx1) + 1.0, 0.0)
            wy = jnp.maximum(
                jnp.minimum(ay2, gy2) - jnp.maximum(ay1, gy1) + 1.0, 0.0)
            inter = wx * wy
            iou = inter / (a1 + a2 - inter)
            gmax = jnp.max(iou)
            lqf = (iou >= gmax - 1e-6) & (gmax >= 0.3)
            better = iou > mo
            mo = jnp.where(better, iou, mo)
            am = jnp.where(better, g, am)
            lq = jnp.where(lqf, g, lq)
        maxov_s[...] = mo
        argm_s[...] = am
        lqgt_s[...] = lq

    @pl.when(s == _NSTEPS)
    def _():
        inside = ((v_ref[...] != 0) & (ax1 >= 0.0) & (ay1 >= 0.0)
                  & (ax2 < _IMG) & (ay2 < _IMG))
        maxov = maxov_s[...]
        argm = argm_s[...]
        lqgt = lqgt_s[...]
        assigned = jnp.where((maxov >= -1.0) & (maxov < 0.3), 0, -1)
        assigned = jnp.where(maxov >= 0.7, argm + 1, assigned)
        assigned = jnp.where(lqgt >= 0, lqgt + 1, assigned)
        assigned = jnp.where(inside, assigned, -1)
        pos_m = assigned > 0
        neg_m = assigned == 0

        rp = rp_ref[...]
        rn = rn_ref[...]

        def bsb(_, st):
            plo, phi, nlo, nhi = st
            pmid = (plo + phi) // 2
            nmid = (nlo + nhi) // 2
            pcnt = jnp.sum(jnp.where(pos_m & (rp <= pmid), 1, 0))
            ncnt = jnp.sum(jnp.where(neg_m & (rn <= nmid), 1, 0))
            pge = pcnt >= _K
            nge = ncnt >= _K
            pc = plo < phi
            nc = nlo < nhi
            return (jnp.where(pc & pge, plo, jnp.where(pc, pmid + 1, plo)),
                    jnp.where(pc & pge, pmid, phi),
                    jnp.where(nc & nge, nlo, jnp.where(nc, nmid + 1, nlo)),
                    jnp.where(nc & nge, nmid, nhi))

        z = jnp.int32(0)
        top = jnp.int32(32768)
        tp, _, tn, _ = jax.lax.fori_loop(0, 16, bsb, (z, top, z, top))
        sp = pos_m & (rp <= tp)
        sn = neg_m & (rn <= tn)

        lab_ref[...] = jnp.where(sp, 1, 0)
        lw_ref[...] = jnp.where(sp | sn, 1.0, 0.0)
        posf_ref[...] = jnp.where(sp, 1.0, 0.0)
        npos_ref[0, 0] = jnp.sum(jnp.where(sp, 1, 0))
        nneg_ref[0, 0] = jnp.sum(jnp.where(sn, 1, 0))

        gidx = jnp.where(lqgt >= 0, lqgt, argm)

        def mgb(i, c):
            x1, y1, x2, y2 = c
            for k in range(_UNROLL):
                j = i * _UNROLL + k
                m = gidx == j
                x1 = jnp.where(m, gt_ref[0, j], x1)
                y1 = jnp.where(m, gt_ref[1, j], y1)
                x2 = jnp.where(m, gt_ref[2, j], x2)
                y2 = jnp.where(m, gt_ref[3, j], y2)
            return (x1, y1, x2, y2)

        zf = jnp.zeros_like(maxov)
        mx1, my1, mx2, my2 = jax.lax.fori_loop(
            0, _G // _UNROLL, mgb, (zf, zf, zf, zf))
        px = (ax1 + ax2) * 0.5
        py = (ay1 + ay2) * 0.5
        pw = ax2 - ax1 + 1.0
        ph = ay2 - ay1 + 1.0
        gx = (mx1 + mx2) * 0.5
        gy = (my1 + my2) * 0.5
        gw = mx2 - mx1 + 1.0
        gh = my2 - my1 + 1.0
        tgt_ref[0] = jnp.where(sp, (gx - px) / pw, 0.0)
        tgt_ref[1] = jnp.where(sp, (gy - py) / ph, 0.0)
        tgt_ref[2] = jnp.where(sp, jnp.log(gw / pw), 0.0)
        tgt_ref[3] = jnp.where(sp, jnp.log(gh / ph), 0.0)


def _run(a4, v2, gt4, rp, rn):
    f32 = jnp.float32
    i32 = jnp.int32
    vmem2 = pl.BlockSpec((_ROWS, _LANES), lambda g: (0, 0))
    return pl.pallas_call(
        _body,
        grid=(_NSTEPS + 1,),
        in_specs=[
            pl.BlockSpec((4, _G), lambda g: (0, 0), memory_space=pltpu.SMEM),
            pl.BlockSpec((4, _ROWS, _LANES), lambda g: (0, 0, 0)),
            vmem2,
            vmem2,
            vmem2,
        ],
        out_specs=[
            vmem2,
            vmem2,
            vmem2,
            pl.BlockSpec((4, _ROWS, _LANES), lambda g: (0, 0, 0)),
            pl.BlockSpec((1, 1), lambda g: (0, 0), memory_space=pltpu.SMEM),
            pl.BlockSpec((1, 1), lambda g: (0, 0), memory_space=pltpu.SMEM),
        ],
        out_shape=[
            jax.ShapeDtypeStruct((_ROWS, _LANES), i32),
            jax.ShapeDtypeStruct((_ROWS, _LANES), f32),
            jax.ShapeDtypeStruct((_ROWS, _LANES), f32),
            jax.ShapeDtypeStruct((4, _ROWS, _LANES), f32),
            jax.ShapeDtypeStruct((1, 1), i32),
            jax.ShapeDtypeStruct((1, 1), i32),
        ],
        scratch_shapes=[
            pltpu.VMEM((_ROWS, _LANES), f32),
            pltpu.VMEM((_ROWS, _LANES), i32),
            pltpu.VMEM((_ROWS, _LANES), i32),
        ],
    )(gt4, a4, v2, rp, rn)


def kernel(anchors, valid_flags, gt_bboxes):
    pad_box = jnp.array([-1e6, -1e6, -1e6 + 100.0, -1e6 + 100.0], jnp.float32)
    a_p = jnp.concatenate(
        [anchors, jnp.broadcast_to(pad_box, (_NP - _N, 4))], axis=0)
    a4 = a_p.T.reshape(4, _ROWS, _LANES)
    v2 = jnp.concatenate(
        [valid_flags.astype(jnp.int32),
         jnp.zeros((_NP - _N,), jnp.int32)]).reshape(_ROWS, _LANES)
    gt4 = gt_bboxes.T
    rp = jnp.asarray(_RANK_POS)
    rn = jnp.asarray(_RANK_NEG)

    lab, lw, posf, tgt, npos, nneg = _run(a4, v2, gt4, rp, rn)

    labels = lab.reshape(-1)[:_N]
    label_weights = lw.reshape(-1)[:_N]
    bbox_targets = tgt.reshape(4, -1)[:, :_N].T
    posf1 = posf.reshape(-1)[:_N]
    bbox_weights = jnp.broadcast_to(posf1[:, None], (_N, 4))
    num_pos = npos[0, 0]
    num_neg = nneg[0, 0]
    return labels, label_weights, bbox_targets, bbox_weights, num_pos, num_neg


# speedup vs baseline: 6.0384x; 1.0310x over previous
import jax
import jax.numpy as jnp
import numpy as np
from jax.experimental import pallas as pl
from jax.experimental.pallas import tpu as pltpu

_N = 20000
_G = 100
_IMG = 1344.0
_ROWS = 160
_LANES = 128
_NP = _ROWS * _LANES
_K = 128


def _make_ranks():
    kp, kn = jax.random.split(jax.random.key(42))
    out = []
    for k in (kp, kn):
        pri = np.asarray(jax.random.uniform(k, (_N,)))
        perm = np.argsort(-pri, kind="stable")
        rank = np.empty(_N, np.int32)
        rank[perm] = np.arange(_N, dtype=np.int32)
        pad = np.full(_NP - _N, np.int32(1 << 30), np.int32)
        out.append(np.concatenate([rank, pad]).reshape(_ROWS, _LANES))
    return out[0], out[1]


_RANK_POS, _RANK_NEG = _make_ranks()


_UNROLL = 50
_NSTEPS = _G // _UNROLL


def _body(gt_ref, a_ref, v_ref, rp_ref, rn_ref,
          lab_ref, lw_ref, posf_ref, tgt_ref, npos_ref, nneg_ref,
          maxov_s, argm_s, lqgt_s):
    s = pl.program_id(0)

    ax1 = a_ref[0]
    ay1 = a_ref[1]
    ax2 = a_ref[2]
    ay2 = a_ref[3]

    @pl.when(s < _NSTEPS)
    def _():
        a1 = (ax2 - ax1 + 1.0) * (ay2 - ay1 + 1.0)
        first = s == 0
        mo = jnp.where(first, jnp.float32(-jnp.inf), maxov_s[...])
        am = jnp.where(first, 0, argm_s[...])
        lq = jnp.where(first, -1, lqgt_s[...])
        for j in range(_UNROLL):
            g = s * _UNROLL + j
            gx1 = gt_ref[0, g]
            gy1 = gt_ref[1, g]
            gx2 = gt_ref[2, g]
            gy2 = gt_ref[3, g]
            a2 = (gx2 - gx1 + 1.0) * (gy2 - gy1 + 1.0)
            wx = jnp.maximum(
                jnp.minimum(ax2, gx2) - jnp.maximum(ax1, gx1) + 1.0, 0.0)
            wy = jnp.maximum(
                jnp.minimum(ay2, gy2) - jnp.maximum(ay1, gy1) + 1.0, 0.0)
            inter = wx * wy
            iou = inter / (a1 + a2 - inter)
            gmax = jnp.max(iou)
            lqf = (iou >= gmax - 1e-6) & (gmax >= 0.3)
            better = iou > mo
            mo = jnp.where(better, iou, mo)
            am = jnp.where(better, g, am)
            lq = jnp.where(lqf, g, lq)
        maxov_s[...] = mo
        argm_s[...] = am
        lqgt_s[...] = lq

    @pl.when(s == _NSTEPS)
    def _():
        inside = ((v_ref[...] != 0) & (ax1 >= 0.0) & (ay1 >= 0.0)
                  & (ax2 < _IMG) & (ay2 < _IMG))
        maxov = maxov_s[...]
        argm = argm_s[...]
        lqgt = lqgt_s[...]
        assigned = jnp.where((maxov >= -1.0) & (maxov < 0.3), 0, -1)
        assigned = jnp.where(maxov >= 0.7, argm + 1, assigned)
        assigned = jnp.where(lqgt >= 0, lqgt + 1, assigned)
        assigned = jnp.where(inside, assigned, -1)
        pos_m = assigned > 0
        neg_m = assigned == 0

        rp = rp_ref[...]
        rn = rn_ref[...]

        def bsb(_, st):
            plo, phi, nlo, nhi = st
            pmid = (plo + phi) // 2
            nmid = (nlo + nhi) // 2
            pcnt = jnp.sum(jnp.where(pos_m & (rp <= pmid), 1, 0))
            ncnt = jnp.sum(jnp.where(neg_m & (rn <= nmid), 1, 0))
            pge = pcnt >= _K
            nge = ncnt >= _K
            pc = plo < phi
            nc = nlo < nhi
            return (jnp.where(pc & pge, plo, jnp.where(pc, pmid + 1, plo)),
                    jnp.where(pc & pge, pmid, phi),
                    jnp.where(nc & nge, nlo, jnp.where(nc, nmid + 1, nlo)),
                    jnp.where(nc & nge, nmid, nhi))

        z = jnp.int32(0)
        top = jnp.int32(32768)
        tp, _, tn, _ = jax.lax.fori_loop(0, 16, bsb, (z, top, z, top))
        sp = pos_m & (rp <= tp)
        sn = neg_m & (rn <= tn)

        lab_ref[...] = jnp.where(sp, 1, 0)
        lw_ref[...] = jnp.where(sp | sn, 1.0, 0.0)
        posf_ref[...] = jnp.where(sp, 1.0, 0.0)
        npos_ref[0, 0] = jnp.sum(jnp.where(sp, 1, 0))
        nneg_ref[0, 0] = jnp.sum(jnp.where(sn, 1, 0))

        gidx = jnp.where(lqgt >= 0, lqgt, argm)

        def mgb(i, c):
            x1, y1, x2, y2 = c
            for k in range(_UNROLL):
                j = i * _UNROLL + k
                m = gidx == j
                x1 = jnp.where(m, gt_ref[0, j], x1)
                y1 = jnp.where(m, gt_ref[1, j], y1)
                x2 = jnp.where(m, gt_ref[2, j], x2)
                y2 = jnp.where(m, gt_ref[3, j], y2)
            return (x1, y1, x2, y2)

        zf = jnp.zeros_like(maxov)
        mx1, my1, mx2, my2 = jax.lax.fori_loop(
            0, _G // _UNROLL, mgb, (zf, zf, zf, zf))
        px = (ax1 + ax2) * 0.5
        py = (ay1 + ay2) * 0.5
        pw = ax2 - ax1 + 1.0
        ph = ay2 - ay1 + 1.0
        gx = (mx1 + mx2) * 0.5
        gy = (my1 + my2) * 0.5
        gw = mx2 - mx1 + 1.0
        gh = my2 - my1 + 1.0
        tgt_ref[0] = jnp.where(sp, (gx - px) / pw, 0.0)
        tgt_ref[1] = jnp.where(sp, (gy - py) / ph, 0.0)
        tgt_ref[2] = jnp.where(sp, jnp.log(gw / pw), 0.0)
        tgt_ref[3] = jnp.where(sp, jnp.log(gh / ph), 0.0)


def _run(a4, v2, gt4, rp, rn):
    f32 = jnp.float32
    i32 = jnp.int32
    vmem2 = pl.BlockSpec((_ROWS, _LANES), lambda g: (0, 0))
    return pl.pallas_call(
        _body,
        grid=(_NSTEPS + 1,),
        in_specs=[
            pl.BlockSpec((4, _G), lambda g: (0, 0), memory_space=pltpu.SMEM),
            pl.BlockSpec((4, _ROWS, _LANES), lambda g: (0, 0, 0)),
            vmem2,
            vmem2,
            vmem2,
        ],
        out_specs=[
            vmem2,
            vmem2,
            vmem2,
            pl.BlockSpec((4, _ROWS, _LANES), lambda g: (0, 0, 0)),
            pl.BlockSpec((1, 1), lambda g: (0, 0), memory_space=pltpu.SMEM),
            pl.BlockSpec((1, 1), lambda g: (0, 0), memory_space=pltpu.SMEM),
        ],
        out_shape=[
            jax.ShapeDtypeStruct((_ROWS, _LANES), i32),
            jax.ShapeDtypeStruct((_ROWS, _LANES), f32),
            jax.ShapeDtypeStruct((_ROWS, _LANES), f32),
            jax.ShapeDtypeStruct((4, _ROWS, _LANES), f32),
            jax.ShapeDtypeStruct((1, 1), i32),
            jax.ShapeDtypeStruct((1, 1), i32),
        ],
        scratch_shapes=[
            pltpu.VMEM((_ROWS, _LANES), f32),
            pltpu.VMEM((_ROWS, _LANES), i32),
            pltpu.VMEM((_ROWS, _LANES), i32),
        ],
    )(gt4, a4, v2, rp, rn)


def kernel(anchors, valid_flags, gt_bboxes):
    pad_box = jnp.array([-1e6, -1e6, -1e6 + 100.0, -1e6 + 100.0], jnp.float32)
    a_p = jnp.concatenate(
        [anchors, jnp.broadcast_to(pad_box, (_NP - _N, 4))], axis=0)
    a4 = a_p.T.reshape(4, _ROWS, _LANES)
    v2 = jnp.concatenate(
        [valid_flags.astype(jnp.int32),
         jnp.zeros((_NP - _N,), jnp.int32)]).reshape(_ROWS, _LANES)
    gt4 = gt_bboxes.T
    rp = jnp.asarray(_RANK_POS)
    rn = jnp.asarray(_RANK_NEG)

    lab, lw, posf, tgt, npos, nneg = _run(a4, v2, gt4, rp, rn)

    labels = lab.reshape(-1)[:_N]
    label_weights = lw.reshape(-1)[:_N]
    bbox_targets = tgt.reshape(4, -1)[:, :_N].T
    posf1 = posf.reshape(-1)[:_N]
    bbox_weights = jnp.broadcast_to(posf1[:, None], (_N, 4))
    num_pos = npos[0, 0]
    num_neg = nneg[0, 0]
    return labels, label_weights, bbox_targets, bbox_weights, num_pos, num_neg


# speedup vs baseline: 6.1261x; 1.0145x over previous
import jax
import jax.numpy as jnp
import numpy as np
from jax.experimental import pallas as pl
from jax.experimental.pallas import tpu as pltpu

_N = 20000
_G = 100
_IMG = 1344.0
_ROWS = 160
_LANES = 128
_NP = _ROWS * _LANES
_K = 128


def _make_ranks():
    kp, kn = jax.random.split(jax.random.key(42))
    out = []
    for k in (kp, kn):
        pri = np.asarray(jax.random.uniform(k, (_N,)))
        perm = np.argsort(-pri, kind="stable")
        rank = np.empty(_N, np.int32)
        rank[perm] = np.arange(_N, dtype=np.int32)
        pad = np.full(_NP - _N, np.int32(1 << 30), np.int32)
        out.append(np.concatenate([rank, pad]).reshape(_ROWS, _LANES))
    return out[0], out[1]


_RANK_POS, _RANK_NEG = _make_ranks()


_UNROLL = 100
_NSTEPS = _G // _UNROLL


def _body(gt_ref, a_ref, v_ref, rp_ref, rn_ref,
          lab_ref, lw_ref, posf_ref, tgt_ref, npos_ref, nneg_ref,
          maxov_s, argm_s, lqgt_s):
    s = pl.program_id(0)

    ax1 = a_ref[0]
    ay1 = a_ref[1]
    ax2 = a_ref[2]
    ay2 = a_ref[3]

    @pl.when(s < _NSTEPS)
    def _():
        a1 = (ax2 - ax1 + 1.0) * (ay2 - ay1 + 1.0)
        first = s == 0
        mo = jnp.where(first, jnp.float32(-jnp.inf), maxov_s[...])
        am = jnp.where(first, 0, argm_s[...])
        lq = jnp.where(first, -1, lqgt_s[...])
        for j in range(_UNROLL):
            g = s * _UNROLL + j
            gx1 = gt_ref[0, g]
            gy1 = gt_ref[1, g]
            gx2 = gt_ref[2, g]
            gy2 = gt_ref[3, g]
            a2 = (gx2 - gx1 + 1.0) * (gy2 - gy1 + 1.0)
            wx = jnp.maximum(
                jnp.minimum(ax2, gx2) - jnp.maximum(ax1, gx1) + 1.0, 0.0)
            wy = jnp.maximum(
                jnp.minimum(ay2, gy2) - jnp.maximum(ay1, gy1) + 1.0, 0.0)
            inter = wx * wy
            iou = inter / (a1 + a2 - inter)
            gmax = jnp.max(iou)
            lqf = (iou >= gmax - 1e-6) & (gmax >= 0.3)
            better = iou > mo
            mo = jnp.where(better, iou, mo)
            am = jnp.where(better, g, am)
            lq = jnp.where(lqf, g, lq)
        maxov_s[...] = mo
        argm_s[...] = am
        lqgt_s[...] = lq

    @pl.when(s == _NSTEPS)
    def _():
        inside = ((v_ref[...] != 0) & (ax1 >= 0.0) & (ay1 >= 0.0)
                  & (ax2 < _IMG) & (ay2 < _IMG))
        maxov = maxov_s[...]
        argm = argm_s[...]
        lqgt = lqgt_s[...]
        assigned = jnp.where((maxov >= -1.0) & (maxov < 0.3), 0, -1)
        assigned = jnp.where(maxov >= 0.7, argm + 1, assigned)
        assigned = jnp.where(lqgt >= 0, lqgt + 1, assigned)
        assigned = jnp.where(inside, assigned, -1)
        pos_m = assigned > 0
        neg_m = assigned == 0

        rp = rp_ref[...]
        rn = rn_ref[...]

        def bsb(_, st):
            plo, phi, nlo, nhi = st
            pmid = (plo + phi) // 2
            nmid = (nlo + nhi) // 2
            pcnt = jnp.sum(jnp.where(pos_m & (rp <= pmid), 1, 0))
            ncnt = jnp.sum(jnp.where(neg_m & (rn <= nmid), 1, 0))
            pge = pcnt >= _K
            nge = ncnt >= _K
            pc = plo < phi
            nc = nlo < nhi
            return (jnp.where(pc & pge, plo, jnp.where(pc, pmid + 1, plo)),
                    jnp.where(pc & pge, pmid, phi),
                    jnp.where(nc & nge, nlo, jnp.where(nc, nmid + 1, nlo)),
                    jnp.where(nc & nge, nmid, nhi))

        z = jnp.int32(0)
        top = jnp.int32(32768)
        tp, _, tn, _ = jax.lax.fori_loop(0, 16, bsb, (z, top, z, top))
        sp = pos_m & (rp <= tp)
        sn = neg_m & (rn <= tn)

        lab_ref[...] = jnp.where(sp, 1, 0)
        lw_ref[...] = jnp.where(sp | sn, 1.0, 0.0)
        posf_ref[...] = jnp.where(sp, 1.0, 0.0)
        npos_ref[0, 0] = jnp.sum(jnp.where(sp, 1, 0))
        nneg_ref[0, 0] = jnp.sum(jnp.where(sn, 1, 0))

        gidx = jnp.where(lqgt >= 0, lqgt, argm)

        def mgb(i, c):
            x1, y1, x2, y2 = c
            for k in range(_UNROLL):
                j = i * _UNROLL + k
                m = gidx == j
                x1 = jnp.where(m, gt_ref[0, j], x1)
                y1 = jnp.where(m, gt_ref[1, j], y1)
                x2 = jnp.where(m, gt_ref[2, j], x2)
                y2 = jnp.where(m, gt_ref[3, j], y2)
            return (x1, y1, x2, y2)

        zf = jnp.zeros_like(maxov)
        mx1, my1, mx2, my2 = jax.lax.fori_loop(
            0, _G // _UNROLL, mgb, (zf, zf, zf, zf))
        px = (ax1 + ax2) * 0.5
        py = (ay1 + ay2) * 0.5
        pw = ax2 - ax1 + 1.0
        ph = ay2 - ay1 + 1.0
        gx = (mx1 + mx2) * 0.5
        gy = (my1 + my2) * 0.5
        gw = mx2 - mx1 + 1.0
        gh = my2 - my1 + 1.0
        tgt_ref[0] = jnp.where(sp, (gx - px) / pw, 0.0)
        tgt_ref[1] = jnp.where(sp, (gy - py) / ph, 0.0)
        tgt_ref[2] = jnp.where(sp, jnp.log(gw / pw), 0.0)
        tgt_ref[3] = jnp.where(sp, jnp.log(gh / ph), 0.0)


def _run(a4, v2, gt4, rp, rn):
    f32 = jnp.float32
    i32 = jnp.int32
    vmem2 = pl.BlockSpec((_ROWS, _LANES), lambda g: (0, 0))
    return pl.pallas_call(
        _body,
        grid=(_NSTEPS + 1,),
        in_specs=[
            pl.BlockSpec((4, _G), lambda g: (0, 0), memory_space=pltpu.SMEM),
            pl.BlockSpec((4, _ROWS, _LANES), lambda g: (0, 0, 0)),
            vmem2,
            vmem2,
            vmem2,
        ],
        out_specs=[
            vmem2,
            vmem2,
            vmem2,
            pl.BlockSpec((4, _ROWS, _LANES), lambda g: (0, 0, 0)),
            pl.BlockSpec((1, 1), lambda g: (0, 0), memory_space=pltpu.SMEM),
            pl.BlockSpec((1, 1), lambda g: (0, 0), memory_space=pltpu.SMEM),
        ],
        out_shape=[
            jax.ShapeDtypeStruct((_ROWS, _LANES), i32),
            jax.ShapeDtypeStruct((_ROWS, _LANES), f32),
            jax.ShapeDtypeStruct((_ROWS, _LANES), f32),
            jax.ShapeDtypeStruct((4, _ROWS, _LANES), f32),
            jax.ShapeDtypeStruct((1, 1), i32),
            jax.ShapeDtypeStruct((1, 1), i32),
        ],
        scratch_shapes=[
            pltpu.VMEM((_ROWS, _LANES), f32),
            pltpu.VMEM((_ROWS, _LANES), i32),
            pltpu.VMEM((_ROWS, _LANES), i32),
        ],
    )(gt4, a4, v2, rp, rn)


def kernel(anchors, valid_flags, gt_bboxes):
    pad_box = jnp.array([-1e6, -1e6, -1e6 + 100.0, -1e6 + 100.0], jnp.float32)
    a_p = jnp.concatenate(
        [anchors, jnp.broadcast_to(pad_box, (_NP - _N, 4))], axis=0)
    a4 = a_p.T.reshape(4, _ROWS, _LANES)
    v2 = jnp.concatenate(
        [valid_flags.astype(jnp.int32),
         jnp.zeros((_NP - _N,), jnp.int32)]).reshape(_ROWS, _LANES)
    gt4 = gt_bboxes.T
    rp = jnp.asarray(_RANK_POS)
    rn = jnp.asarray(_RANK_NEG)

    lab, lw, posf, tgt, npos, nneg = _run(a4, v2, gt4, rp, rn)

    labels = lab.reshape(-1)[:_N]
    label_weights = lw.reshape(-1)[:_N]
    bbox_targets = tgt.reshape(4, -1)[:, :_N].T
    posf1 = posf.reshape(-1)[:_N]
    bbox_weights = jnp.broadcast_to(posf1[:, None], (_N, 4))
    num_pos = npos[0, 0]
    num_neg = nneg[0, 0]
    return labels, label_weights, bbox_targets, bbox_weights, num_pos, num_neg
